# Initial kernel scaffold; baseline (speedup 1.0000x reference)
import jax
import jax.numpy as jnp
from jax.experimental import pallas as pl


def kernel(x, start_pos, freqs_cos, freqs_sin, wq, wk, wv, wo):
    raise NotImplementedError("write your pallas kernel here")


# speedup vs baseline: 1.7463x; 1.7463x over previous
import math

import jax
import jax.numpy as jnp
import numpy as np
from jax.experimental import pallas as pl
from jax.experimental.pallas import tpu as pltpu

BLOCK_SIZE = 32
LOCAL_BLOCKS = 2
SINK_NUM = 4
WINDOW = LOCAL_BLOCKS * BLOCK_SIZE
S = 2048
DIM = 768
N_HEADS = 12
N_KV_HEADS = 4
N_REP = N_HEADS // N_KV_HEADS
HEAD_DIM = 64
HALF = HEAD_DIM // 2
KV_DIM = N_KV_HEADS * HEAD_DIM

TQ = 256
TK = TQ + WINDOW
TSINK = 32
NEG = float(np.finfo(np.float32).min)
SCALE = 1.0 / math.sqrt(HEAD_DIM)


def _matmul_kernel(x_ref, w_ref, o_ref):
    o_ref[...] = jnp.dot(x_ref[...], w_ref[...],
                         preferred_element_type=jnp.float32)


def _rot_half(t):
    return jnp.concatenate([-t[:, HALF:], t[:, :HALF]], axis=1)


def _attn_kernel(q_ref, k_ref, v_ref, cs_ref, sn_ref, o_ref):
    i = pl.program_id(0)
    q0 = pl.multiple_of(i * TQ, TQ)
    ks = pl.multiple_of(jnp.maximum(q0 - WINDOW, 0), WINDOW)

    csq = cs_ref[pl.ds(q0, TQ), :]
    snq = sn_ref[pl.ds(q0, TQ), :]
    csk = cs_ref[pl.ds(ks, TK), :]
    snk = sn_ref[pl.ds(ks, TK), :]
    css = cs_ref[0:TSINK, :]
    sns = sn_ref[0:TSINK, :]

    a_w = q0 + jax.lax.broadcasted_iota(jnp.int32, (TQ, TK), 0)
    jw = ks + jax.lax.broadcasted_iota(jnp.int32, (TQ, TK), 1)
    m_w = (jw <= a_w) & ((jw >= a_w - WINDOW) | (jw < SINK_NUM))

    a_s = q0 + jax.lax.broadcasted_iota(jnp.int32, (TQ, TSINK), 0)
    js = jax.lax.broadcasted_iota(jnp.int32, (TQ, TSINK), 1)
    m_s = (js < SINK_NUM) & (js < a_s) & (js < ks)

    kwr, ksr, vw, vs = [], [], [], []
    for g in range(N_KV_HEADS):
        c = slice(g * HEAD_DIM, (g + 1) * HEAD_DIM)
        kw = k_ref[pl.ds(ks, TK), c]
        kwr.append(kw * csk + _rot_half(kw) * snk)
        ksk = k_ref[0:TSINK, c]
        ksr.append(ksk * css + _rot_half(ksk) * sns)
        vw.append(v_ref[pl.ds(ks, TK), c])
        vs.append(v_ref[0:TSINK, c])

    outs = []
    for h in range(N_HEADS):
        g = h // N_REP
        c = slice(h * HEAD_DIM, (h + 1) * HEAD_DIM)
        q = q_ref[:, c] * SCALE
        qr = q * csq + _rot_half(q) * snq

        sw = jax.lax.dot_general(qr, kwr[g], (((1,), (1,)), ((), ())),
                                 preferred_element_type=jnp.float32)
        ss = jax.lax.dot_general(qr, ksr[g], (((1,), (1,)), ((), ())),
                                 preferred_element_type=jnp.float32)
        sw = jnp.where(m_w, sw, NEG)
        ss = jnp.where(m_s, ss, NEG)

        s = jnp.concatenate([ss, sw], axis=1)
        m = jnp.max(s, axis=1, keepdims=True)
        p = jnp.exp(s - m)
        p = p / jnp.sum(p, axis=1, keepdims=True)

        vcat = jnp.concatenate([vs[g], vw[g]], axis=0)
        outs.append(jnp.dot(p, vcat, preferred_element_type=jnp.float32))

    o_ref[...] = jnp.concatenate(outs, axis=1)


def kernel(x, start_pos, freqs_cos, freqs_sin, wq, wk, wv, wo):
    del start_pos
    x2 = x[0]

    perm = np.concatenate([np.arange(0, HEAD_DIM, 2), np.arange(1, HEAD_DIM, 2)])
    wq_p = wq.reshape(DIM, N_HEADS, HEAD_DIM)[:, :, perm].reshape(DIM, N_HEADS * HEAD_DIM)
    wk_p = wk.reshape(DIM, N_KV_HEADS, HEAD_DIM)[:, :, perm].reshape(DIM, KV_DIM)
    wcat = jnp.concatenate([wq_p, wk_p, wv], axis=1)

    cs = jnp.concatenate([freqs_cos, freqs_cos], axis=1)
    sn = jnp.concatenate([freqs_sin, freqs_sin], axis=1)

    nrow = S // TQ
    qkv = pl.pallas_call(
        _matmul_kernel,
        grid=(nrow,),
        in_specs=[
            pl.BlockSpec((TQ, DIM), lambda r: (r, 0)),
            pl.BlockSpec((DIM, DIM + 2 * KV_DIM), lambda r: (0, 0)),
        ],
        out_specs=pl.BlockSpec((TQ, DIM + 2 * KV_DIM), lambda r: (r, 0)),
        out_shape=jax.ShapeDtypeStruct((S, DIM + 2 * KV_DIM), jnp.float32),
    )(x2, wcat)

    q = qkv[:, :DIM]
    k = qkv[:, DIM:DIM + KV_DIM]
    v = qkv[:, DIM + KV_DIM:]

    attn = pl.pallas_call(
        _attn_kernel,
        grid=(nrow,),
        in_specs=[
            pl.BlockSpec((TQ, DIM), lambda r: (r, 0)),
            pl.BlockSpec((S, KV_DIM), lambda r: (0, 0)),
            pl.BlockSpec((S, KV_DIM), lambda r: (0, 0)),
            pl.BlockSpec((S, HEAD_DIM), lambda r: (0, 0)),
            pl.BlockSpec((S, HEAD_DIM), lambda r: (0, 0)),
        ],
        out_specs=pl.BlockSpec((TQ, DIM), lambda r: (r, 0)),
        out_shape=jax.ShapeDtypeStruct((S, DIM), jnp.float32),
    )(q, k, v, cs, sn)

    out = pl.pallas_call(
        _matmul_kernel,
        grid=(nrow,),
        in_specs=[
            pl.BlockSpec((TQ, DIM), lambda r: (r, 0)),
            pl.BlockSpec((DIM, DIM), lambda r: (0, 0)),
        ],
        out_specs=pl.BlockSpec((TQ, DIM), lambda r: (r, 0)),
        out_shape=jax.ShapeDtypeStruct((S, DIM), jnp.float32),
    )(attn, wo)

    return out[None, :, :]


# speedup vs baseline: 2.2295x; 1.2767x over previous
import math

import jax
import jax.numpy as jnp
import numpy as np
from jax.experimental import pallas as pl
from jax.experimental.pallas import tpu as pltpu

BLOCK_SIZE = 32
LOCAL_BLOCKS = 2
SINK_NUM = 4
WINDOW = LOCAL_BLOCKS * BLOCK_SIZE
S = 2048
DIM = 768
N_HEADS = 12
N_KV_HEADS = 4
N_REP = N_HEADS // N_KV_HEADS
HEAD_DIM = 64
HALF = HEAD_DIM // 2
KV_DIM = N_KV_HEADS * HEAD_DIM
QKV_DIM = DIM + 2 * KV_DIM

TQ = 256
TK = TQ + WINDOW
TSINK = 32
NEG = float(np.finfo(np.float32).min)
SCALE = 1.0 / math.sqrt(HEAD_DIM)


def _rot_half(t):
    return jnp.concatenate([-t[:, HALF:], t[:, :HALF]], axis=1)


def _fused_kernel(x_ref, w_ref, wo_ref, cs_ref, sn_ref, o_ref, kscr, vscr):
    i = pl.program_id(0)
    q0 = pl.multiple_of(i * TQ, TQ)
    ks = pl.multiple_of(jnp.maximum(q0 - WINDOW, 0), WINDOW)

    qkv = jnp.dot(x_ref[...], w_ref[...], preferred_element_type=jnp.float32)
    q = qkv[:, :DIM]
    k = qkv[:, DIM:DIM + KV_DIM]
    v = qkv[:, DIM + KV_DIM:]

    csq = cs_ref[...]
    snq = sn_ref[...]
    cs4 = jnp.concatenate([csq] * N_KV_HEADS, axis=1)
    sn4 = jnp.concatenate([snq] * N_KV_HEADS, axis=1)

    krot = jnp.concatenate(
        [_rot_half(k[:, g * HEAD_DIM:(g + 1) * HEAD_DIM]) for g in range(N_KV_HEADS)],
        axis=1)
    kr = k * cs4 + krot * sn4
    kscr[pl.ds(q0, TQ), :] = kr
    vscr[pl.ds(q0, TQ), :] = v

    @pl.when(i == 0)
    def _():
        kscr[pl.ds(TQ, WINDOW), :] = jnp.zeros((WINDOW, KV_DIM), jnp.float32)
        vscr[pl.ds(TQ, WINDOW), :] = jnp.zeros((WINDOW, KV_DIM), jnp.float32)

    a_w = q0 + jax.lax.broadcasted_iota(jnp.int32, (TQ, TK), 0)
    jw = ks + jax.lax.broadcasted_iota(jnp.int32, (TQ, TK), 1)
    m_w = (jw <= a_w) & ((jw >= a_w - WINDOW) | (jw < SINK_NUM))

    a_s = q0 + jax.lax.broadcasted_iota(jnp.int32, (TQ, TSINK), 0)
    js = jax.lax.broadcasted_iota(jnp.int32, (TQ, TSINK), 1)
    m_s = (js < SINK_NUM) & (js < a_s) & (js < ks)

    kwin, ksink, vwin, vsink = [], [], [], []
    for g in range(N_KV_HEADS):
        c = slice(g * HEAD_DIM, (g + 1) * HEAD_DIM)
        kwin.append(kscr[pl.ds(ks, TK), c])
        ksink.append(kscr[0:TSINK, c])
        vwin.append(vscr[pl.ds(ks, TK), c])
        vsink.append(vscr[0:TSINK, c])

    outs = []
    for h in range(N_HEADS):
        g = h // N_REP
        c = slice(h * HEAD_DIM, (h + 1) * HEAD_DIM)
        qh = q[:, c] * SCALE
        qr = qh * csq + _rot_half(qh) * snq

        sw = jax.lax.dot_general(qr, kwin[g], (((1,), (1,)), ((), ())),
                                 preferred_element_type=jnp.float32)
        ss = jax.lax.dot_general(qr, ksink[g], (((1,), (1,)), ((), ())),
                                 preferred_element_type=jnp.float32)
        sw = jnp.where(m_w, sw, NEG)
        ss = jnp.where(m_s, ss, NEG)

        s = jnp.concatenate([ss, sw], axis=1)
        m = jnp.max(s, axis=1, keepdims=True)
        p = jnp.exp(s - m)
        p = p / jnp.sum(p, axis=1, keepdims=True)

        vcat = jnp.concatenate([vsink[g], vwin[g]], axis=0)
        outs.append(jnp.dot(p, vcat, preferred_element_type=jnp.float32))

    attn = jnp.concatenate(outs, axis=1)
    o_ref[...] = jnp.dot(attn, wo_ref[...], preferred_element_type=jnp.float32)


def kernel(x, start_pos, freqs_cos, freqs_sin, wq, wk, wv, wo):
    del start_pos
    x2 = x[0]

    perm = np.concatenate([np.arange(0, HEAD_DIM, 2), np.arange(1, HEAD_DIM, 2)])
    wq_p = wq.reshape(DIM, N_HEADS, HEAD_DIM)[:, :, perm].reshape(DIM, N_HEADS * HEAD_DIM)
    wk_p = wk.reshape(DIM, N_KV_HEADS, HEAD_DIM)[:, :, perm].reshape(DIM, KV_DIM)
    wcat = jnp.concatenate([wq_p, wk_p, wv], axis=1)

    cs = jnp.concatenate([freqs_cos, freqs_cos], axis=1)
    sn = jnp.concatenate([freqs_sin, freqs_sin], axis=1)

    nrow = S // TQ
    out = pl.pallas_call(
        _fused_kernel,
        grid=(nrow,),
        in_specs=[
            pl.BlockSpec((TQ, DIM), lambda r: (r, 0)),
            pl.BlockSpec((DIM, QKV_DIM), lambda r: (0, 0)),
            pl.BlockSpec((DIM, DIM), lambda r: (0, 0)),
            pl.BlockSpec((TQ, HEAD_DIM), lambda r: (r, 0)),
            pl.BlockSpec((TQ, HEAD_DIM), lambda r: (r, 0)),
        ],
        out_specs=pl.BlockSpec((TQ, DIM), lambda r: (r, 0)),
        out_shape=jax.ShapeDtypeStruct((S, DIM), jnp.float32),
        scratch_shapes=[
            pltpu.VMEM((S, KV_DIM), jnp.float32),
            pltpu.VMEM((S, KV_DIM), jnp.float32),
        ],
        compiler_params=pltpu.CompilerParams(
            dimension_semantics=("arbitrary",),
        ),
    )(x2, wcat, wo, cs, sn)

    return out[None, :, :]


# speedup vs baseline: 3.4505x; 1.5477x over previous
import math

import jax
import jax.numpy as jnp
import numpy as np
from jax.experimental import pallas as pl
from jax.experimental.pallas import tpu as pltpu

BLOCK_SIZE = 32
LOCAL_BLOCKS = 2
SINK_NUM = 4
WINDOW = LOCAL_BLOCKS * BLOCK_SIZE
S = 2048
DIM = 768
N_HEADS = 12
N_KV_HEADS = 4
N_REP = N_HEADS // N_KV_HEADS
HEAD_DIM = 64
HALF = HEAD_DIM // 2
KV_DIM = N_KV_HEADS * HEAD_DIM
Q_DIM = N_HEADS * HEAD_DIM
PROJ_DIM = 2 * Q_DIM + 3 * KV_DIM

TQ = 256
TK = TQ + WINDOW
TSINK = 32
TC = TSINK + TK
NEG = -1e30
SCALE = 1.0 / math.sqrt(HEAD_DIM)


def _fused_kernel(x_ref, w_ref, wo_ref, cs_ref, sn_ref, bias_ref,
                  o_ref, kscr, vscr):
    i = pl.program_id(0)
    q0 = pl.multiple_of(i * TQ, TQ)
    ks = pl.multiple_of(jnp.maximum(q0 - WINDOW, 0), WINDOW)

    proj = jnp.dot(x_ref[...], w_ref[...], preferred_element_type=jnp.float32)
    q = proj[:, :Q_DIM]
    qrot = proj[:, Q_DIM:2 * Q_DIM]
    k = proj[:, 2 * Q_DIM:2 * Q_DIM + KV_DIM]
    krot = proj[:, 2 * Q_DIM + KV_DIM:2 * Q_DIM + 2 * KV_DIM]
    v = proj[:, 2 * Q_DIM + 2 * KV_DIM:]

    csq = cs_ref[...]
    snq = sn_ref[...]
    cs12 = jnp.concatenate([csq] * N_HEADS, axis=1)
    sn12 = jnp.concatenate([snq] * N_HEADS, axis=1)
    cs4 = jnp.concatenate([csq] * N_KV_HEADS, axis=1)
    sn4 = jnp.concatenate([snq] * N_KV_HEADS, axis=1)

    qr = (q * cs12 + qrot * sn12) * SCALE
    kr = k * cs4 + krot * sn4
    kscr[pl.ds(q0, TQ), :] = kr
    vscr[pl.ds(q0, TQ), :] = v

    @pl.when(i == 0)
    def _():
        kscr[pl.ds(TQ, WINDOW), :] = jnp.zeros((WINDOW, KV_DIM), jnp.float32)
        vscr[pl.ds(TQ, WINDOW), :] = jnp.zeros((WINDOW, KV_DIM), jnp.float32)

    bias = bias_ref[0]

    kcat, vcat = [], []
    for g in range(N_KV_HEADS):
        c = slice(g * HEAD_DIM, (g + 1) * HEAD_DIM)
        kcat.append(jnp.concatenate([kscr[0:TSINK, c], kscr[pl.ds(ks, TK), c]],
                                    axis=0))
        vcat.append(jnp.concatenate([vscr[0:TSINK, c], vscr[pl.ds(ks, TK), c]],
                                    axis=0))

    outs = []
    for h in range(N_HEADS):
        g = h // N_REP
        c = slice(h * HEAD_DIM, (h + 1) * HEAD_DIM)
        s = jax.lax.dot_general(qr[:, c], kcat[g], (((1,), (1,)), ((), ())),
                                preferred_element_type=jnp.float32)
        p = jnp.exp(s + bias)
        pv = jnp.dot(p, vcat[g], preferred_element_type=jnp.float32)
        denom = jnp.sum(p, axis=1, keepdims=True)
        outs.append(pv / denom)

    attn = jnp.concatenate(outs, axis=1)
    o_ref[...] = jnp.dot(attn, wo_ref[...], preferred_element_type=jnp.float32)


def _mask_bias():
    r = np.arange(TQ)[:, None]
    cs_ = np.arange(TSINK)[None, :]
    cw = np.arange(TK)[None, :]
    sink0 = np.zeros((TQ, TSINK), bool)
    jw0 = cw
    win0 = (jw0 <= r) & ((jw0 >= r - WINDOW) | (jw0 < SINK_NUM))
    sink1 = np.broadcast_to(cs_ < SINK_NUM, (TQ, TSINK))
    win1 = (cw - WINDOW <= r) & (cw >= r)
    m = np.stack([np.concatenate([sink0, win0], axis=1),
                  np.concatenate([sink1, win1], axis=1)])
    return jnp.asarray(np.where(m, 0.0, NEG), dtype=jnp.float32)


def kernel(x, start_pos, freqs_cos, freqs_sin, wq, wk, wv, wo):
    del start_pos
    x2 = x[0]

    perm = np.concatenate([np.arange(0, HEAD_DIM, 2), np.arange(1, HEAD_DIM, 2)])
    rotp = np.concatenate([np.arange(HALF, HEAD_DIM), np.arange(0, HALF)])
    sgn = np.concatenate([-np.ones(HALF), np.ones(HALF)]).astype(np.float32)

    wq_p = wq.reshape(DIM, N_HEADS, HEAD_DIM)[:, :, perm]
    wk_p = wk.reshape(DIM, N_KV_HEADS, HEAD_DIM)[:, :, perm]
    wq_r = wq_p[:, :, rotp] * sgn
    wk_r = wk_p[:, :, rotp] * sgn
    wcat = jnp.concatenate([
        wq_p.reshape(DIM, Q_DIM), wq_r.reshape(DIM, Q_DIM),
        wk_p.reshape(DIM, KV_DIM), wk_r.reshape(DIM, KV_DIM), wv], axis=1)

    cs = jnp.concatenate([freqs_cos, freqs_cos], axis=1)
    sn = jnp.concatenate([freqs_sin, freqs_sin], axis=1)
    bias = _mask_bias()

    nrow = S // TQ
    out = pl.pallas_call(
        _fused_kernel,
        grid=(nrow,),
        in_specs=[
            pl.BlockSpec((TQ, DIM), lambda r: (r, 0)),
            pl.BlockSpec((DIM, PROJ_DIM), lambda r: (0, 0)),
            pl.BlockSpec((DIM, DIM), lambda r: (0, 0)),
            pl.BlockSpec((TQ, HEAD_DIM), lambda r: (r, 0)),
            pl.BlockSpec((TQ, HEAD_DIM), lambda r: (r, 0)),
            pl.BlockSpec((1, TQ, TC), lambda r: (jnp.minimum(r, 1), 0, 0)),
        ],
        out_specs=pl.BlockSpec((TQ, DIM), lambda r: (r, 0)),
        out_shape=jax.ShapeDtypeStruct((S, DIM), jnp.float32),
        scratch_shapes=[
            pltpu.VMEM((S, KV_DIM), jnp.float32),
            pltpu.VMEM((S, KV_DIM), jnp.float32),
        ],
        compiler_params=pltpu.CompilerParams(
            dimension_semantics=("arbitrary",),
        ),
    )(x2, wcat, wo, cs, sn, bias)

    return out[None, :, :]


# speedup vs baseline: 5.9684x; 1.7297x over previous
import math

import jax
import jax.numpy as jnp
import numpy as np
from jax.experimental import pallas as pl
from jax.experimental.pallas import tpu as pltpu

BLOCK_SIZE = 32
LOCAL_BLOCKS = 2
SINK_NUM = 4
WINDOW = LOCAL_BLOCKS * BLOCK_SIZE
S = 2048
DIM = 768
N_HEADS = 12
N_KV_HEADS = 4
N_REP = N_HEADS // N_KV_HEADS
HEAD_DIM = 64
KV_DIM = N_KV_HEADS * HEAD_DIM
Q_DIM = N_HEADS * HEAD_DIM

TQ = 256
TK = TQ + WINDOW
TSINK = 32
TC = TSINK + TK
NEG = -1e30
SCALE = 1.0 / math.sqrt(HEAD_DIM)


def _rot_pairs(t):
    r1 = jnp.roll(t, 1, axis=1)
    rm = jnp.roll(t, -1, axis=1)
    lane = jax.lax.broadcasted_iota(jnp.int32, t.shape, 1)
    return jnp.where(lane % 2 == 0, -rm, r1)


def _fused_kernel(x_ref, wq_ref, wk_ref, wv_ref, wo_ref, ci_ref, si_ref,
                  bias_ref, o_ref, kscr, vscr):
    i = pl.program_id(0)
    q0 = pl.multiple_of(i * TQ, TQ)
    ks = pl.multiple_of(jnp.maximum(q0 - WINDOW, 0), WINDOW)

    xt = x_ref[...]
    q = jnp.dot(xt, wq_ref[...], preferred_element_type=jnp.float32)
    k = jnp.dot(xt, wk_ref[...], preferred_element_type=jnp.float32)
    v = jnp.dot(xt, wv_ref[...], preferred_element_type=jnp.float32)

    ci = ci_ref[...]
    si = si_ref[...]
    ci12 = jnp.concatenate([ci] * N_HEADS, axis=1)
    si12 = jnp.concatenate([si] * N_HEADS, axis=1)
    ci4 = jnp.concatenate([ci] * N_KV_HEADS, axis=1)
    si4 = jnp.concatenate([si] * N_KV_HEADS, axis=1)

    qr = (q * ci12 + _rot_pairs(q) * si12) * SCALE
    kr = k * ci4 + _rot_pairs(k) * si4
    kscr[pl.ds(q0, TQ), :] = kr
    vscr[pl.ds(q0, TQ), :] = v

    @pl.when(i == 0)
    def _():
        kscr[pl.ds(TQ, WINDOW), :] = jnp.zeros((WINDOW, KV_DIM), jnp.float32)
        vscr[pl.ds(TQ, WINDOW), :] = jnp.zeros((WINDOW, KV_DIM), jnp.float32)

    bias = bias_ref[0]

    kcat, vcat = [], []
    for g in range(N_KV_HEADS):
        c = slice(g * HEAD_DIM, (g + 1) * HEAD_DIM)
        kcat.append(jnp.concatenate([kscr[0:TSINK, c], kscr[pl.ds(ks, TK), c]],
                                    axis=0))
        vcat.append(jnp.concatenate([vscr[0:TSINK, c], vscr[pl.ds(ks, TK), c]],
                                    axis=0))

    outs = []
    for h in range(N_HEADS):
        g = h // N_REP
        c = slice(h * HEAD_DIM, (h + 1) * HEAD_DIM)
        s = jax.lax.dot_general(qr[:, c], kcat[g], (((1,), (1,)), ((), ())),
                                preferred_element_type=jnp.float32)
        p = jnp.exp(s + bias)
        pv = jnp.dot(p, vcat[g], preferred_element_type=jnp.float32)
        denom = jnp.sum(p, axis=1, keepdims=True)
        outs.append(pv / denom)

    attn = jnp.concatenate(outs, axis=1)
    o_ref[...] = jnp.dot(attn, wo_ref[...], preferred_element_type=jnp.float32)


def _mask_bias():
    r = np.arange(TQ)[:, None]
    cs_ = np.arange(TSINK)[None, :]
    cw = np.arange(TK)[None, :]
    sink0 = np.zeros((TQ, TSINK), bool)
    win0 = (cw <= r) & ((cw >= r - WINDOW) | (cw < SINK_NUM))
    sink1 = np.broadcast_to(cs_ < SINK_NUM, (TQ, TSINK))
    win1 = (cw - WINDOW <= r) & (cw >= r)
    m = np.stack([np.concatenate([sink0, win0], axis=1),
                  np.concatenate([sink1, win1], axis=1)])
    return jnp.asarray(np.where(m, 0.0, NEG), dtype=jnp.float32)


def kernel(x, start_pos, freqs_cos, freqs_sin, wq, wk, wv, wo):
    del start_pos
    x2 = x[0]

    ci = jnp.repeat(freqs_cos, 2, axis=1)
    si = jnp.repeat(freqs_sin, 2, axis=1)
    bias = _mask_bias()

    nrow = S // TQ
    out = pl.pallas_call(
        _fused_kernel,
        grid=(nrow,),
        in_specs=[
            pl.BlockSpec((TQ, DIM), lambda r: (r, 0)),
            pl.BlockSpec((DIM, Q_DIM), lambda r: (0, 0)),
            pl.BlockSpec((DIM, KV_DIM), lambda r: (0, 0)),
            pl.BlockSpec((DIM, KV_DIM), lambda r: (0, 0)),
            pl.BlockSpec((DIM, DIM), lambda r: (0, 0)),
            pl.BlockSpec((TQ, HEAD_DIM), lambda r: (r, 0)),
            pl.BlockSpec((TQ, HEAD_DIM), lambda r: (r, 0)),
            pl.BlockSpec((1, TQ, TC), lambda r: (jnp.minimum(r, 1), 0, 0)),
        ],
        out_specs=pl.BlockSpec((TQ, DIM), lambda r: (r, 0)),
        out_shape=jax.ShapeDtypeStruct((S, DIM), jnp.float32),
        scratch_shapes=[
            pltpu.VMEM((S, KV_DIM), jnp.float32),
            pltpu.VMEM((S, KV_DIM), jnp.float32),
        ],
        compiler_params=pltpu.CompilerParams(
            dimension_semantics=("arbitrary",),
        ),
    )(x2, wq, wk, wv, wo, ci, si, bias)

    return out[None, :, :]
